# Initial kernel scaffold; baseline (speedup 1.0000x reference)
import jax
import jax.numpy as jnp
from jax.experimental import pallas as pl


def kernel(x_tre, x_sea, x_res):
    raise NotImplementedError("write your pallas kernel here")


# speedup vs baseline: 1.0985x; 1.0985x over previous
import numpy as np
import jax
import jax.numpy as jnp
from jax.experimental import pallas as pl
from jax.experimental.pallas import tpu as pltpu

_MASKING_RATE = 0.4
_MSK_SCALAR = 0.0
_CHUNK = 256


def _rank_mask_kernel(n_mask, r_ref, o_ref):
    i = pl.program_id(0)
    t = r_ref.shape[1]
    r = r_ref[0, :]
    rows = r_ref[0, pl.ds(i * _CHUNK, _CHUNK)]
    rj = r[None, :]
    rt = rows[:, None]
    jidx = jax.lax.broadcasted_iota(jnp.int32, (_CHUNK, t), 1)
    tidx = i * _CHUNK + jax.lax.broadcasted_iota(jnp.int32, (_CHUNK, t), 0)
    before = (rj < rt) | ((rj == rt) & (jidx < tidx))
    ranks = jnp.sum(before.astype(jnp.int32), axis=1)
    o_ref[0, :] = (ranks < n_mask).astype(jnp.float32)


def _apply_kernel(m_ref, x1_ref, x2_ref, x3_ref, o1_ref, o2_ref, o3_ref):
    keep = m_ref[0, 0, :][:, None] == 0.0
    o1_ref[0] = jnp.where(keep, x1_ref[0], _MSK_SCALAR)
    o2_ref[0] = jnp.where(keep, x2_ref[0], _MSK_SCALAR)
    o3_ref[0] = jnp.where(keep, x3_ref[0], _MSK_SCALAR)


def kernel(x_tre, x_sea, x_res):
    b, t, f = x_tre.shape
    n_mask = int(np.ceil(t * _MASKING_RATE))
    rk = jax.random.key(42)
    r = jax.random.uniform(rk, (t,), minval=0.0, maxval=1.0)

    mask = pl.pallas_call(
        lambda r_ref, o_ref: _rank_mask_kernel(n_mask, r_ref, o_ref),
        grid=(t // _CHUNK,),
        in_specs=[pl.BlockSpec((1, t), lambda i: (0, 0))],
        out_specs=pl.BlockSpec((1, _CHUNK), lambda i: (0, i)),
        out_shape=jax.ShapeDtypeStruct((1, t), jnp.float32),
    )(r[None, :])

    z_masks = mask[0] != 0.0

    tb = 512
    m3 = mask.reshape(t // tb, 1, tb)
    x_spec = pl.BlockSpec((1, tb, f), lambda bi, ti: (bi, ti, 0))
    m_spec = pl.BlockSpec((1, 1, tb), lambda bi, ti: (ti, 0, 0))
    z_tre, z_sea, z_res = pl.pallas_call(
        _apply_kernel,
        grid=(b, t // tb),
        in_specs=[m_spec, x_spec, x_spec, x_spec],
        out_specs=[x_spec, x_spec, x_spec],
        out_shape=[jax.ShapeDtypeStruct((b, t, f), jnp.float32)] * 3,
        compiler_params=pltpu.CompilerParams(
            dimension_semantics=("parallel", "parallel"),
        ),
    )(m3, x_tre, x_sea, x_res)

    return (z_tre, z_sea, z_res, z_masks)


# speedup vs baseline: 1.2064x; 1.0982x over previous
import numpy as np
import jax
import jax.numpy as jnp
from jax.experimental import pallas as pl
from jax.experimental.pallas import tpu as pltpu

_MASKING_RATE = 0.4
_MSK_SCALAR = 0.0


def _fused_kernel(n_mask, r_ref, x1_ref, x2_ref, x3_ref,
                  o1_ref, o2_ref, o3_ref, m_ref):
    ti = pl.program_id(1)
    t = r_ref.shape[1]
    tb = x1_ref.shape[1]
    r = r_ref[0, :]
    rows = r_ref[0, pl.ds(ti * tb, tb)]
    rj = r[None, :]
    rt = rows[:, None]
    jidx = jax.lax.broadcasted_iota(jnp.int32, (tb, t), 1)
    tidx = ti * tb + jax.lax.broadcasted_iota(jnp.int32, (tb, t), 0)
    before = (rj < rt) | ((rj == rt) & (jidx < tidx))
    ranks = jnp.sum(before.astype(jnp.int32), axis=1, keepdims=True)
    masked = ranks < n_mask
    m_ref[0, :] = masked.astype(jnp.float32)[:, 0]
    o1_ref[0] = jnp.where(masked, _MSK_SCALAR, x1_ref[0])
    o2_ref[0] = jnp.where(masked, _MSK_SCALAR, x2_ref[0])
    o3_ref[0] = jnp.where(masked, _MSK_SCALAR, x3_ref[0])


def kernel(x_tre, x_sea, x_res):
    b, t, f = x_tre.shape
    n_mask = int(np.ceil(t * _MASKING_RATE))
    rk = jax.random.key(42)
    r = jax.random.uniform(rk, (t,), minval=0.0, maxval=1.0)

    tb = 512
    x_spec = pl.BlockSpec((1, tb, f), lambda bi, ti: (bi, ti, 0))
    r_spec = pl.BlockSpec((1, t), lambda bi, ti: (0, 0))
    m_spec = pl.BlockSpec((1, tb), lambda bi, ti: (0, ti))
    z_tre, z_sea, z_res, mask = pl.pallas_call(
        lambda *refs: _fused_kernel(n_mask, *refs),
        grid=(b, t // tb),
        in_specs=[r_spec, x_spec, x_spec, x_spec],
        out_specs=[x_spec, x_spec, x_spec, m_spec],
        out_shape=[jax.ShapeDtypeStruct((b, t, f), jnp.float32)] * 3
        + [jax.ShapeDtypeStruct((1, t), jnp.float32)],
        compiler_params=pltpu.CompilerParams(
            dimension_semantics=("arbitrary", "arbitrary"),
        ),
    )(r[None, :], x_tre, x_sea, x_res)

    return (z_tre, z_sea, z_res, mask[0] != 0.0)


# speedup vs baseline: 1.2172x; 1.0090x over previous
import numpy as np
import jax
import jax.numpy as jnp
from jax.experimental import pallas as pl
from jax.experimental.pallas import tpu as pltpu

_MASKING_RATE = 0.4
_MSK_SCALAR = 0.0


def _fused_kernel(n_mask, r_ref, x1_ref, x2_ref, x3_ref,
                  o1_ref, o2_ref, o3_ref, m_ref):
    ti = pl.program_id(1)
    t = r_ref.shape[1]
    tb = x1_ref.shape[1]
    r = r_ref[0, :]
    rows = r_ref[0, pl.ds(ti * tb, tb)]
    rj = r[None, :]
    rt = rows[:, None]
    jidx = jax.lax.broadcasted_iota(jnp.int32, (tb, t), 1)
    tidx = ti * tb + jax.lax.broadcasted_iota(jnp.int32, (tb, t), 0)
    before = (rj < rt) | ((rj == rt) & (jidx < tidx))
    ranks = jnp.sum(before.astype(jnp.int32), axis=1, keepdims=True)
    masked = ranks < n_mask
    m_ref[0, :] = masked.astype(jnp.float32)[:, 0]
    o1_ref[0] = jnp.where(masked, _MSK_SCALAR, x1_ref[0])
    o2_ref[0] = jnp.where(masked, _MSK_SCALAR, x2_ref[0])
    o3_ref[0] = jnp.where(masked, _MSK_SCALAR, x3_ref[0])


def kernel(x_tre, x_sea, x_res):
    b, t, f = x_tre.shape
    n_mask = int(np.ceil(t * _MASKING_RATE))
    rk = jax.random.key(42)
    r = jax.random.uniform(rk, (t,), minval=0.0, maxval=1.0)

    tb = 1024
    x_spec = pl.BlockSpec((1, tb, f), lambda bi, ti: (bi, ti, 0))
    r_spec = pl.BlockSpec((1, t), lambda bi, ti: (0, 0))
    m_spec = pl.BlockSpec((1, tb), lambda bi, ti: (0, ti))
    z_tre, z_sea, z_res, mask = pl.pallas_call(
        lambda *refs: _fused_kernel(n_mask, *refs),
        grid=(b, t // tb),
        in_specs=[r_spec, x_spec, x_spec, x_spec],
        out_specs=[x_spec, x_spec, x_spec, m_spec],
        out_shape=[jax.ShapeDtypeStruct((b, t, f), jnp.float32)] * 3
        + [jax.ShapeDtypeStruct((1, t), jnp.float32)],
        compiler_params=pltpu.CompilerParams(
            dimension_semantics=("arbitrary", "arbitrary"),
        ),
    )(r[None, :], x_tre, x_sea, x_res)

    return (z_tre, z_sea, z_res, mask[0] != 0.0)
